# baseline (speedup 1.0000x reference)
import functools

import jax
import jax.numpy as jnp
from jax import lax
from jax.experimental import pallas as pl
from jax.experimental.pallas import tpu as pltpu
from jax.experimental.pallas import tpu_sc as plsc

_LANES = 16


def _coeff_body(n_idx, n_pad_cols, alpha_hbm, beta_hbm, idx_hbm,
                a_hbm, b_hbm, a_v, b_v, idx_v, s_v):
    c = lax.axis_index("c")
    s = lax.axis_index("s")

    @pl.when(jnp.logical_and(c == 0, s == 0))
    def _():
        pltpu.sync_copy(alpha_hbm, s_v.at[pl.ds(0, _LANES)])
        pltpu.sync_copy(beta_hbm, s_v.at[pl.ds(_LANES, _LANES)])
        pltpu.sync_copy(idx_hbm, idx_v)
        ones = jnp.ones((_LANES,), jnp.float32)
        for i in range(n_pad_cols // _LANES):
            a_v[pl.ds(i * _LANES, _LANES)] = ones
            b_v[pl.ds(i * _LANES, _LANES)] = ones
        av = s_v[pl.ds(0, _LANES)]
        bv = s_v[pl.ds(_LANES, _LANES)]
        lane = lax.iota(jnp.int32, _LANES)
        for k in range(idx_v.shape[0] // _LANES):
            idxv = idx_v[pl.ds(k * _LANES, _LANES)]
            mask = (lane + (k * _LANES)) < n_idx
            plsc.store_scatter(a_v, [idxv], av, mask=mask)
            plsc.store_scatter(b_v, [idxv], bv, mask=mask)
        pltpu.sync_copy(a_v, a_hbm)
        pltpu.sync_copy(b_v, b_hbm)


def _affine_body(x_ref, a_ref, b_ref, o_ref):
    o_ref[...] = x_ref[...] * a_ref[...] + b_ref[...]


@functools.partial(jax.jit, static_argnames=("row_block",))
def _bias_layer(x, alpha, beta, clss, row_block=512):
    n_rows, n_cols = x.shape
    n_idx = clss.shape[0]
    n_pad_cols = (n_cols + 127) // 128 * 128
    n_pad_idx = (n_idx + _LANES - 1) // _LANES * _LANES

    alpha16 = jnp.broadcast_to(alpha.astype(jnp.float32), (_LANES,))
    beta16 = jnp.broadcast_to(beta.astype(jnp.float32), (_LANES,))
    idx_pad = jnp.pad(clss.astype(jnp.int32), (0, n_pad_idx - n_idx))

    mesh = plsc.VectorSubcoreMesh(core_axis_name="c", subcore_axis_name="s")
    a_pad, b_pad = pl.kernel(
        functools.partial(_coeff_body, n_idx, n_pad_cols),
        out_type=(jax.ShapeDtypeStruct((n_pad_cols,), jnp.float32),
                  jax.ShapeDtypeStruct((n_pad_cols,), jnp.float32)),
        scratch_types=[
            pltpu.VMEM((n_pad_cols,), jnp.float32),
            pltpu.VMEM((n_pad_cols,), jnp.float32),
            pltpu.VMEM((n_pad_idx,), jnp.int32),
            pltpu.VMEM((2 * _LANES,), jnp.float32),
        ],
        mesh=mesh,
        compiler_params=pltpu.CompilerParams(needs_layout_passes=False),
        name="sc_coeff_scatter",
    )(alpha16, beta16, idx_pad)

    a_row = a_pad[:n_cols].reshape(1, n_cols)
    b_row = b_pad[:n_cols].reshape(1, n_cols)

    grid = (n_rows // row_block,)
    return pl.pallas_call(
        _affine_body,
        grid=grid,
        in_specs=[
            pl.BlockSpec((row_block, n_cols), lambda i: (i, 0)),
            pl.BlockSpec((1, n_cols), lambda i: (0, 0)),
            pl.BlockSpec((1, n_cols), lambda i: (0, 0)),
        ],
        out_specs=pl.BlockSpec((row_block, n_cols), lambda i: (i, 0)),
        out_shape=jax.ShapeDtypeStruct((n_rows, n_cols), x.dtype),
        compiler_params=pltpu.CompilerParams(
            dimension_semantics=("arbitrary",),
        ),
        name="tc_affine",
    )(x, a_row, b_row)


def kernel(x, alpha, beta, clss):
    return _bias_layer(x, alpha, beta, clss)
